# baseline (speedup 1.0000x reference)
import math

import jax
import jax.numpy as jnp
import numpy as np
from jax import lax
from jax.experimental import pallas as pl
from jax.experimental.pallas import tpu as pltpu
from jax.experimental.pallas import tpu_sc as plsc

_D_MODEL = 768
_S = 32
_W = 256
_CTX = 3 * _W
_LANES = 16
_CHUNKS = _D_MODEL // _LANES


def _build_pe():
    pe = np.zeros((_CTX, _D_MODEL), dtype=np.float32)
    position = np.arange(0, _CTX, dtype=np.float32)[:, None]
    div_term = np.exp(
        np.arange(0, _D_MODEL, 2, dtype=np.float32) * (-math.log(10000.0) / _D_MODEL)
    )
    pe[:, 0::2] = np.sin(position * div_term)
    pe[:, 1::2] = np.cos(position * div_term)
    return pe


_PE = _build_pe()


def _top3_sc_kernel(fr_hbm, ft_hbm, out_hbm, fr_v, ft_v, d_v, idx_v):
    wid = lax.axis_index("s") * 2 + lax.axis_index("c")
    pltpu.sync_copy(fr_hbm, fr_v)
    pltpu.sync_copy(ft_hbm, ft_v)

    def h_step(hh, carry):
        d0v, d1v = carry
        fwc = fr_v[pl.ds(wid * _D_MODEL + hh * _LANES, _LANES)]
        for u in range(_LANES):
            h = hh * _LANES + u
            fw = fwc[u]
            r0 = ft_v[pl.ds(h * _S, _LANES)]
            r1 = ft_v[pl.ds(h * _S + _LANES, _LANES)]
            d0v = d0v + jnp.abs(r0 - fw)
            d1v = d1v + jnp.abs(r1 - fw)
        return (d0v, d1v)

    zeros = jnp.zeros((_LANES,), jnp.float32)
    d0v, d1v = lax.fori_loop(0, _CHUNKS, h_step, (zeros, zeros))
    inf = jnp.float32(np.inf)
    d_v[pl.ds(0, _LANES)] = d0v
    d_v[pl.ds(_LANES, _LANES)] = d1v
    d_v[pl.ds(2 * _LANES, _LANES)] = jnp.full((_LANES,), inf, jnp.float32)

    def j_step(j, carry):
        d0, d1, d2, i0, i1, i2 = carry
        d = d_v[pl.ds(j, _LANES)][0]
        lt0 = d < d0
        lt1 = d < d1
        lt2 = d < d2
        n_d0 = jnp.where(lt0, d, d0)
        n_i0 = jnp.where(lt0, j, i0)
        n_d1 = jnp.where(lt0, d0, jnp.where(lt1, d, d1))
        n_i1 = jnp.where(lt0, i0, jnp.where(lt1, j, i1))
        n_d2 = jnp.where(lt1, d1, jnp.where(lt2, d, d2))
        n_i2 = jnp.where(lt1, i1, jnp.where(lt2, j, i2))
        return (n_d0, n_d1, n_d2, n_i0, n_i1, n_i2)

    carry = lax.fori_loop(
        0, _S, j_step, (inf, inf, inf, jnp.int32(0), jnp.int32(0), jnp.int32(0))
    )
    _, _, _, i0, i1, i2 = carry
    iota = lax.iota(jnp.int32, _LANES)
    vec = jnp.where(
        iota == 0, i0, jnp.where(iota == 1, i1, jnp.where(iota == 2, i2, 0))
    )
    idx_v[...] = vec
    pltpu.sync_copy(idx_v, out_hbm.at[wid])


def _attn_kernel(idx_ref, a_ref, b_ref, c_ref, pe_ref, o_ref):
    del idx_ref
    pe = pe_ref[...]
    a = a_ref[0] + pe[:_W]
    b = b_ref[0] + pe[_W : 2 * _W]
    c = c_ref[0] + pe[2 * _W :]
    q = a
    dn = (((1,), (1,)), ((), ()))
    s = jnp.concatenate(
        [
            jax.lax.dot_general(q, a, dn, preferred_element_type=jnp.float32),
            jax.lax.dot_general(q, b, dn, preferred_element_type=jnp.float32),
            jax.lax.dot_general(q, c, dn, preferred_element_type=jnp.float32),
        ],
        axis=1,
    ) * jnp.float32(1.0 / math.sqrt(_D_MODEL))
    m = jnp.max(s, axis=1, keepdims=True)
    e = jnp.exp(s - m)
    p = e / jnp.sum(e, axis=1, keepdims=True)
    o = (
        jnp.dot(p[:, :_W], a, preferred_element_type=jnp.float32)
        + jnp.dot(p[:, _W : 2 * _W], b, preferred_element_type=jnp.float32)
        + jnp.dot(p[:, 2 * _W :], c, preferred_element_type=jnp.float32)
    )
    o_ref[0] = o[: _W - 1]


def kernel(inputs):
    first = inputs[:, 0, :]
    first_r = first.reshape(-1)
    first_t = first.T.reshape(-1)
    mesh = plsc.VectorSubcoreMesh(core_axis_name="c", subcore_axis_name="s")
    top3 = pl.kernel(
        _top3_sc_kernel,
        out_type=jax.ShapeDtypeStruct((_S, _LANES), jnp.int32),
        mesh=mesh,
        scratch_types=[
            pltpu.VMEM((_S * _D_MODEL,), jnp.float32),
            pltpu.VMEM((_S * _D_MODEL,), jnp.float32),
            pltpu.VMEM((3 * _LANES,), jnp.float32),
            pltpu.VMEM((_LANES,), jnp.int32),
        ],
    )(first_r, first_t)

    grid_spec = pltpu.PrefetchScalarGridSpec(
        num_scalar_prefetch=1,
        grid=(_S,),
        in_specs=[
            pl.BlockSpec((1, _W, _D_MODEL), lambda i, idx: (idx[i, 0], 0, 0)),
            pl.BlockSpec((1, _W, _D_MODEL), lambda i, idx: (idx[i, 1], 0, 0)),
            pl.BlockSpec((1, _W, _D_MODEL), lambda i, idx: (idx[i, 2], 0, 0)),
            pl.BlockSpec((_CTX, _D_MODEL), lambda i, idx: (0, 0)),
        ],
        out_specs=pl.BlockSpec((1, _W - 1, _D_MODEL), lambda i, idx: (i, 0, 0)),
    )
    out = pl.pallas_call(
        _attn_kernel,
        grid_spec=grid_spec,
        out_shape=jax.ShapeDtypeStruct((_S, _W - 1, _D_MODEL), jnp.float32),
    )(top3, inputs, inputs, inputs, _PE)
    return out
